# Initial kernel scaffold; baseline (speedup 1.0000x reference)
import jax
import jax.numpy as jnp
from jax.experimental import pallas as pl


def kernel(prefix, embedding):
    raise NotImplementedError("write your pallas kernel here")


# speedup vs baseline: 1.8505x; 1.8505x over previous
import functools

import jax
import jax.numpy as jnp
from jax import lax
from jax.experimental import pallas as pl
from jax.experimental.pallas import tpu as pltpu
from jax.experimental.pallas import tpu_sc as plsc

PRE_SEQ_LEN = 128
LAYER_NUM = 32
HEAD_NUM_KV = 8
SIZE_PER_HEAD = 128
EMB_DIM = LAYER_NUM * SIZE_PER_HEAD * HEAD_NUM_KV * 2
BATCH = 8

CHUNKS = EMB_DIM // SIZE_PER_HEAD
NUM_TASKS = BATCH * CHUNKS
NUM_WORKERS = 32
TASKS_PER_WORKER = NUM_TASKS // NUM_WORKERS
WORKERS_PER_B = CHUNKS // TASKS_PER_WORKER
LANES = 16

_mesh = plsc.VectorSubcoreMesh(core_axis_name="core", subcore_axis_name="subcore")


@functools.partial(
    pl.kernel,
    out_type=jax.ShapeDtypeStruct((NUM_TASKS, PRE_SEQ_LEN, SIZE_PER_HEAD), jnp.float32),
    mesh=_mesh,
    scratch_types=[
        pltpu.VMEM((PRE_SEQ_LEN,), jnp.int32),
        pltpu.VMEM((PRE_SEQ_LEN,), jnp.int32),
        pltpu.VMEM((PRE_SEQ_LEN, SIZE_PER_HEAD), jnp.float32),
        pltpu.SemaphoreType.DMA,
    ],
)
def _gather_kernel(table, prefix, out, base_v, idx_v, rows_v, sem):
    wid = lax.axis_index("subcore") * 2 + lax.axis_index("core")
    b = wid // WORKERS_PER_B
    c0 = (wid % WORKERS_PER_B) * TASKS_PER_WORKER
    t0 = wid * TASKS_PER_WORKER

    pltpu.sync_copy(prefix.at[b], base_v)
    for i in range(PRE_SEQ_LEN // LANES):
        sl = pl.ds(i * LANES, LANES)
        base_v[sl] = base_v[sl] * CHUNKS

    def body(j, carry):
        c = c0 + j
        for i in range(PRE_SEQ_LEN // LANES):
            sl = pl.ds(i * LANES, LANES)
            idx_v[sl] = base_v[sl] + c
        pltpu.async_copy(table.at[idx_v], rows_v, sem).wait()
        pltpu.sync_copy(rows_v, out.at[t0 + j])
        return carry

    lax.fori_loop(0, TASKS_PER_WORKER, body, 0)


def kernel(prefix, embedding):
    table = embedding.reshape(PRE_SEQ_LEN * CHUNKS, SIZE_PER_HEAD)
    out = _gather_kernel(table, prefix)
    return out.reshape(BATCH, LAYER_NUM * 2, HEAD_NUM_KV, PRE_SEQ_LEN, SIZE_PER_HEAD)


# speedup vs baseline: 2.7084x; 1.4636x over previous
import functools

import jax
import jax.numpy as jnp
from jax import lax
from jax.experimental import pallas as pl
from jax.experimental.pallas import tpu as pltpu
from jax.experimental.pallas import tpu_sc as plsc

PRE_SEQ_LEN = 128
LAYER_NUM = 32
HEAD_NUM_KV = 8
SIZE_PER_HEAD = 128
EMB_DIM = LAYER_NUM * SIZE_PER_HEAD * HEAD_NUM_KV * 2
BATCH = 8

CHUNKS = EMB_DIM // SIZE_PER_HEAD
NUM_TASKS = BATCH * CHUNKS
NUM_WORKERS = 32
TASKS_PER_WORKER = NUM_TASKS // NUM_WORKERS
WORKERS_PER_B = CHUNKS // TASKS_PER_WORKER
LANES = 16

NBUF = 4
LAG = 2
ROUNDS = TASKS_PER_WORKER // NBUF

_mesh = plsc.VectorSubcoreMesh(core_axis_name="core", subcore_axis_name="subcore")


@functools.partial(
    pl.kernel,
    out_type=jax.ShapeDtypeStruct((NUM_TASKS, PRE_SEQ_LEN, SIZE_PER_HEAD), jnp.float32),
    mesh=_mesh,
    scratch_types=[
        pltpu.VMEM((PRE_SEQ_LEN,), jnp.int32),
        pltpu.VMEM((NBUF, PRE_SEQ_LEN), jnp.int32),
        pltpu.VMEM((NBUF, PRE_SEQ_LEN, SIZE_PER_HEAD), jnp.float32),
    ]
    + [pltpu.SemaphoreType.DMA] * (2 * NBUF),
)
def _gather_kernel(table, prefix, out, base_v, idx_v, rows_v, *sems):
    gsem = sems[:NBUF]
    osem = sems[NBUF:]
    wid = lax.axis_index("subcore") * 2 + lax.axis_index("core")
    b = wid // WORKERS_PER_B
    c0 = (wid % WORKERS_PER_B) * TASKS_PER_WORKER
    t0 = wid * TASKS_PER_WORKER

    pltpu.sync_copy(prefix.at[b], base_v)
    for i in range(PRE_SEQ_LEN // LANES):
        sl = pl.ds(i * LANES, LANES)
        base_v[sl] = base_v[sl] * CHUNKS

    def fire_gather(buf, c):
        for i in range(PRE_SEQ_LEN // LANES):
            sl = pl.ds(i * LANES, LANES)
            idx_v[buf, sl] = base_v[sl] + c
        pltpu.async_copy(table.at[idx_v.at[buf]], rows_v.at[buf], gsem[buf])

    def wait_gather(buf):
        pltpu.make_async_copy(table.at[idx_v.at[buf]], rows_v.at[buf], gsem[buf]).wait()

    def fire_out(buf, t):
        pltpu.async_copy(rows_v.at[buf], out.at[t], osem[buf])

    def wait_out(buf, t):
        pltpu.make_async_copy(rows_v.at[buf], out.at[t], osem[buf]).wait()

    for j in range(NBUF):
        fire_gather(j, c0 + j)
    for j in range(NBUF - LAG):
        wait_gather(j)
        fire_out(j, t0 + j)

    def round_body(r, carry):
        jr = r * NBUF
        for bb in range(NBUF):
            j = jr + bb
            wait_out(bb, t0 + j - NBUF)
            fire_gather(bb, c0 + j)
            b2 = (bb - LAG) % NBUF
            wait_gather(b2)
            fire_out(b2, t0 + j - LAG)
        return carry

    lax.fori_loop(1, ROUNDS, round_body, 0)

    last = TASKS_PER_WORKER
    for j in range(last - LAG, last):
        buf = j % NBUF
        wait_gather(buf)
        fire_out(buf, t0 + j)
    for j in range(last - NBUF, last):
        buf = j % NBUF
        wait_out(buf, t0 + j)


def kernel(prefix, embedding):
    table = embedding.reshape(PRE_SEQ_LEN * CHUNKS, SIZE_PER_HEAD)
    out = _gather_kernel(table, prefix)
    return out.reshape(BATCH, LAYER_NUM * 2, HEAD_NUM_KV, PRE_SEQ_LEN, SIZE_PER_HEAD)


# speedup vs baseline: 2.7786x; 1.0259x over previous
import functools

import jax
import jax.numpy as jnp
from jax import lax
from jax.experimental import pallas as pl
from jax.experimental.pallas import tpu as pltpu
from jax.experimental.pallas import tpu_sc as plsc

PRE_SEQ_LEN = 128
LAYER_NUM = 32
HEAD_NUM_KV = 8
SIZE_PER_HEAD = 128
EMB_DIM = LAYER_NUM * SIZE_PER_HEAD * HEAD_NUM_KV * 2
BATCH = 8

CHUNKS = EMB_DIM // SIZE_PER_HEAD
NUM_TASKS = BATCH * CHUNKS
NUM_WORKERS = 32
TASKS_PER_WORKER = NUM_TASKS // NUM_WORKERS
WORKERS_PER_B = CHUNKS // TASKS_PER_WORKER
LANES = 16

NBUF = 6
LAG = 3
ROUNDS = TASKS_PER_WORKER // NBUF
TAIL = TASKS_PER_WORKER - ROUNDS * NBUF

_mesh = plsc.VectorSubcoreMesh(core_axis_name="core", subcore_axis_name="subcore")


@functools.partial(
    pl.kernel,
    out_type=jax.ShapeDtypeStruct((NUM_TASKS, PRE_SEQ_LEN, SIZE_PER_HEAD), jnp.float32),
    mesh=_mesh,
    scratch_types=[
        pltpu.VMEM((PRE_SEQ_LEN,), jnp.int32),
        pltpu.VMEM((NBUF, PRE_SEQ_LEN), jnp.int32),
        pltpu.VMEM((NBUF, PRE_SEQ_LEN, SIZE_PER_HEAD), jnp.float32),
    ]
    + [pltpu.SemaphoreType.DMA] * (2 * NBUF),
)
def _gather_kernel(table, prefix, out, base_v, idx_v, rows_v, *sems):
    gsem = sems[:NBUF]
    osem = sems[NBUF:]
    wid = lax.axis_index("subcore") * 2 + lax.axis_index("core")
    b = wid // WORKERS_PER_B
    c0 = (wid % WORKERS_PER_B) * TASKS_PER_WORKER
    t0 = wid * TASKS_PER_WORKER

    pltpu.sync_copy(prefix.at[b], base_v)
    for i in range(PRE_SEQ_LEN // LANES):
        sl = pl.ds(i * LANES, LANES)
        base_v[sl] = base_v[sl] * CHUNKS

    def fire_gather(buf, c):
        for i in range(PRE_SEQ_LEN // LANES):
            sl = pl.ds(i * LANES, LANES)
            idx_v[buf, sl] = base_v[sl] + c
        pltpu.async_copy(table.at[idx_v.at[buf]], rows_v.at[buf], gsem[buf])

    def wait_gather(buf):
        pltpu.make_async_copy(table.at[idx_v.at[buf]], rows_v.at[buf], gsem[buf]).wait()

    def fire_out(buf, t):
        pltpu.async_copy(rows_v.at[buf], out.at[t], osem[buf])

    def wait_out(buf, t):
        pltpu.make_async_copy(rows_v.at[buf], out.at[t], osem[buf]).wait()

    for j in range(NBUF):
        fire_gather(j, c0 + j)
    for j in range(NBUF - LAG):
        wait_gather(j)
        fire_out(j, t0 + j)

    def round_body(r, carry):
        jr = r * NBUF
        for bb in range(NBUF):
            j = jr + bb
            wait_out(bb, t0 + j - NBUF)
            fire_gather(bb, c0 + j)
            b2 = (bb - LAG) % NBUF
            wait_gather(b2)
            fire_out(b2, t0 + j - LAG)
        return carry

    lax.fori_loop(1, ROUNDS, round_body, 0)

    jt = ROUNDS * NBUF
    for bb in range(TAIL):
        j = jt + bb
        wait_out(bb, t0 + j - NBUF)
        fire_gather(bb, c0 + j)
        b2 = (bb - LAG) % NBUF
        wait_gather(b2)
        fire_out(b2, t0 + j - LAG)

    last = TASKS_PER_WORKER
    for j in range(last - LAG, last):
        buf = j % NBUF
        wait_gather(buf)
        fire_out(buf, t0 + j)
    for j in range(last - NBUF, last):
        buf = j % NBUF
        wait_out(buf, t0 + j)


def kernel(prefix, embedding):
    table = embedding.reshape(PRE_SEQ_LEN * CHUNKS, SIZE_PER_HEAD)
    out = _gather_kernel(table, prefix)
    return out.reshape(BATCH, LAYER_NUM * 2, HEAD_NUM_KV, PRE_SEQ_LEN, SIZE_PER_HEAD)


# speedup vs baseline: 2.8493x; 1.0255x over previous
import functools

import jax
import jax.numpy as jnp
from jax import lax
from jax.experimental import pallas as pl
from jax.experimental.pallas import tpu as pltpu
from jax.experimental.pallas import tpu_sc as plsc

PRE_SEQ_LEN = 128
LAYER_NUM = 32
HEAD_NUM_KV = 8
SIZE_PER_HEAD = 128
EMB_DIM = LAYER_NUM * SIZE_PER_HEAD * HEAD_NUM_KV * 2
BATCH = 8

CHUNKS = EMB_DIM // SIZE_PER_HEAD
NUM_TASKS = BATCH * CHUNKS
LANES = 16

PCH = 16
PHASES = (CHUNKS // 2) // PCH
TPP = PCH // 2
NBUF = 4
ROUNDS = TPP // NBUF
R_PER_TILE = PRE_SEQ_LEN // 16

_mesh = plsc.VectorSubcoreMesh(core_axis_name="core", subcore_axis_name="subcore")


@functools.partial(
    pl.kernel,
    out_type=jax.ShapeDtypeStruct((NUM_TASKS, PRE_SEQ_LEN, SIZE_PER_HEAD), jnp.float32),
    mesh=_mesh,
    scratch_types=[
        pltpu.VMEM((PRE_SEQ_LEN,), jnp.int32),
        pltpu.VMEM((PRE_SEQ_LEN,), jnp.int32),
        pltpu.VMEM((NBUF, PRE_SEQ_LEN, SIZE_PER_HEAD), jnp.float32),
        pltpu.VMEM_SHARED((2, PRE_SEQ_LEN * PCH, SIZE_PER_HEAD), jnp.float32),
        pltpu.SemaphoreType.DMA,
        pltpu.SemaphoreType.DMA,
    ]
    + [pltpu.SemaphoreType.DMA] * NBUF,
)
def _gather_kernel(table, prefix, out, base_v, idx_v, rows_v, stage, gsem, ssem, *osem):
    core = lax.axis_index("core")
    sid = lax.axis_index("subcore")
    b = sid // 2
    half = sid % 2
    c_sc = core * (CHUNKS // 2)

    pltpu.sync_copy(prefix.at[b], base_v)
    for i in range(PRE_SEQ_LEN // LANES):
        sl = pl.ds(i * LANES, LANES)
        base_v[sl] = base_v[sl] * PCH

    def fire_staging(p):
        nb = p % 2
        cstart = c_sc + p * PCH
        for k in range(R_PER_TILE):
            r = sid * R_PER_TILE + k
            pltpu.async_copy(
                table.at[r, pl.ds(cstart, PCH)], stage.at[nb, pl.ds(r * PCH, PCH)], ssem
            )

    def wait_staging(p):
        nb = p % 2
        cstart = c_sc + p * PCH
        for k in range(R_PER_TILE):
            r = sid * R_PER_TILE + k
            pltpu.make_async_copy(
                table.at[r, pl.ds(cstart, PCH)], stage.at[nb, pl.ds(r * PCH, PCH)], ssem
            ).wait()

    def out_row(p, k):
        return b * CHUNKS + c_sc + p * PCH + half * TPP + k

    def fire_out(buf, t):
        pltpu.async_copy(rows_v.at[buf], out.at[t], osem[buf])

    def wait_out(buf, t):
        pltpu.make_async_copy(rows_v.at[buf], out.at[t], osem[buf]).wait()

    fire_staging(0)
    wait_staging(0)
    plsc.subcore_barrier()

    def phase_body(p, carry):
        @pl.when(p < PHASES - 1)
        def _():
            fire_staging(p + 1)

        nb = p % 2
        sbuf = stage.at[nb]

        def round_body(r, carry2):
            for bb in range(NBUF):
                k = r * NBUF + bb
                cl = half * TPP + k
                j = p * TPP + k

                @pl.when(j >= NBUF)
                def _():
                    wait_out(bb, out_row(p, k))

                for i in range(PRE_SEQ_LEN // LANES):
                    sl = pl.ds(i * LANES, LANES)
                    idx_v[sl] = base_v[sl] + cl
                pltpu.async_copy(sbuf.at[idx_v], rows_v.at[bb], gsem).wait()
                fire_out(bb, out_row(p, k))
            return carry2

        lax.fori_loop(0, ROUNDS, round_body, 0)

        @pl.when(p < PHASES - 1)
        def _():
            wait_staging(p + 1)

        plsc.subcore_barrier()
        return carry

    lax.fori_loop(0, PHASES, phase_body, 0)

    for bb in range(NBUF):
        wait_out(bb, out_row(PHASES - 1, TPP - NBUF + bb))


def kernel(prefix, embedding):
    table = embedding.reshape(PRE_SEQ_LEN, CHUNKS, SIZE_PER_HEAD)
    out = _gather_kernel(table, prefix)
    return out.reshape(BATCH, LAYER_NUM * 2, HEAD_NUM_KV, PRE_SEQ_LEN, SIZE_PER_HEAD)
